# baseline (speedup 1.0000x reference)
import functools

import jax
import jax.numpy as jnp
from jax import lax
from jax.experimental import pallas as pl
from jax.experimental.pallas import tpu as pltpu
from jax.experimental.pallas import tpu_sc as plsc

NC = 2
NS = 16
NW = NC * NS
LANES = 16
CHUNK = 16
NBUF = 2
LOOKAHEAD = 1


def _make_sc_kernel(N, E, V):
    rows_per_worker = N // NW
    n_chunks = rows_per_worker // CHUNK
    mesh = plsc.VectorSubcoreMesh(core_axis_name="c", subcore_axis_name="s")

    @functools.partial(
        pl.kernel,
        out_type=jax.ShapeDtypeStruct((N, E), jnp.float32),
        mesh=mesh,
        scratch_types=[
            pltpu.VMEM((n_chunks, CHUNK), jnp.int32),
            pltpu.VMEM((NBUF, CHUNK, E), jnp.float32),
            pltpu.VMEM((NBUF, CHUNK, E), jnp.float32),
            pltpu.VMEM((NBUF, CHUNK, E), jnp.float32),
            pltpu.SemaphoreType.DMA((NBUF,)),
            pltpu.SemaphoreType.DMA((NBUF,)),
            pltpu.SemaphoreType.DMA((NBUF,)),
        ],
    )
    def body(tok_hbm, idx_hbm, table_hbm, out_hbm, idx_v, gat_v, tok_v,
             o_v, gsem, tsem, osem):
        wid = lax.axis_index("s") * NC + lax.axis_index("c")
        base = wid * rows_per_worker
        pltpu.sync_copy(idx_hbm.at[wid], idx_v)

        def start_in(i, b):
            pltpu.async_copy(table_hbm.at[idx_v.at[i]], gat_v.at[b],
                             gsem.at[b])
            pltpu.async_copy(tok_hbm.at[pl.ds(base + i * CHUNK, CHUNK), :],
                             tok_v.at[b], tsem.at[b])

        def wait_in(i, b):
            pltpu.make_async_copy(table_hbm.at[idx_v.at[i]], gat_v.at[b],
                                  gsem.at[b]).wait()
            pltpu.make_async_copy(
                tok_hbm.at[pl.ds(base + i * CHUNK, CHUNK), :],
                tok_v.at[b], tsem.at[b]).wait()

        def start_out(i, b):
            pltpu.async_copy(o_v.at[b],
                             out_hbm.at[pl.ds(base + i * CHUNK, CHUNK), :],
                             osem.at[b])

        def wait_out(i, b):
            pltpu.make_async_copy(
                o_v.at[b],
                out_hbm.at[pl.ds(base + i * CHUNK, CHUNK), :],
                osem.at[b]).wait()

        for b in range(LOOKAHEAD):
            start_in(b, b)

        @pl.loop(0, n_chunks, step=NBUF)
        def pair(i0):
            for b in range(NBUF):
                i = i0 + b
                wait_in(i, b)

                @pl.when(i + LOOKAHEAD < n_chunks)
                def _():
                    start_in(i + LOOKAHEAD, (b + LOOKAHEAD) % NBUF)

                @pl.when(i >= NBUF)
                def _():
                    wait_out(i - NBUF, b)

                for r in range(CHUNK):
                    @plsc.parallel_loop(0, E // LANES, unroll=8)
                    def col(j):
                        sl = pl.ds(j * LANES, LANES)
                        o_v[b, r, sl] = gat_v[b, r, sl] + tok_v[b, r, sl]

                start_out(i, b)

        for k in range(NBUF):
            wait_out(n_chunks - NBUF + k, (n_chunks - NBUF + k) % NBUF)

    return body


def kernel(token_embedding, pos, pos_table):
    B, L, E = token_embedding.shape
    V = pos_table.shape[0]
    N = B * L
    tok = token_embedding.reshape(N, E)
    idx = pos.reshape(NW, N // (NW * CHUNK), CHUNK).astype(jnp.int32)
    out = _make_sc_kernel(N, E, V)(tok, idx, pos_table)
    return out.reshape(B, L, E)


# speedup vs baseline: 1.0484x; 1.0484x over previous
import functools

import jax
import jax.numpy as jnp
from jax import lax
from jax.experimental import pallas as pl
from jax.experimental.pallas import tpu as pltpu
from jax.experimental.pallas import tpu_sc as plsc

NC = 2
NS = 16
NW = NC * NS
LANES = 16
CHUNK = 8
NBUF = 4
LOOKAHEAD = 3


def _make_sc_kernel(N, E, V):
    rows_per_worker = N // NW
    n_chunks = rows_per_worker // CHUNK
    mesh = plsc.VectorSubcoreMesh(core_axis_name="c", subcore_axis_name="s")

    @functools.partial(
        pl.kernel,
        out_type=jax.ShapeDtypeStruct((N, E), jnp.float32),
        mesh=mesh,
        scratch_types=[
            pltpu.VMEM((n_chunks, CHUNK), jnp.int32),
            pltpu.VMEM((NBUF, CHUNK, E), jnp.float32),
            pltpu.VMEM((NBUF, CHUNK, E), jnp.float32),
            pltpu.VMEM((NBUF, CHUNK, E), jnp.float32),
            pltpu.SemaphoreType.DMA((NBUF,)),
            pltpu.SemaphoreType.DMA((NBUF,)),
            pltpu.SemaphoreType.DMA((NBUF,)),
        ],
    )
    def body(tok_hbm, idx_hbm, table_hbm, out_hbm, idx_v, gat_v, tok_v,
             o_v, gsem, tsem, osem):
        wid = lax.axis_index("s") * NC + lax.axis_index("c")
        base = wid * rows_per_worker
        pltpu.sync_copy(idx_hbm.at[wid], idx_v)

        def start_in(i, b):
            pltpu.async_copy(table_hbm.at[idx_v.at[i]], gat_v.at[b],
                             gsem.at[b])
            pltpu.async_copy(tok_hbm.at[pl.ds(base + i * CHUNK, CHUNK), :],
                             tok_v.at[b], tsem.at[b])

        def wait_in(i, b):
            pltpu.make_async_copy(table_hbm.at[idx_v.at[i]], gat_v.at[b],
                                  gsem.at[b]).wait()
            pltpu.make_async_copy(
                tok_hbm.at[pl.ds(base + i * CHUNK, CHUNK), :],
                tok_v.at[b], tsem.at[b]).wait()

        def start_out(i, b):
            pltpu.async_copy(o_v.at[b],
                             out_hbm.at[pl.ds(base + i * CHUNK, CHUNK), :],
                             osem.at[b])

        def wait_out(i, b):
            pltpu.make_async_copy(
                o_v.at[b],
                out_hbm.at[pl.ds(base + i * CHUNK, CHUNK), :],
                osem.at[b]).wait()

        for b in range(LOOKAHEAD):
            start_in(b, b)

        @pl.loop(0, n_chunks, step=NBUF)
        def quad(i0):
            for b in range(NBUF):
                i = i0 + b
                wait_in(i, b)

                @pl.when(i + LOOKAHEAD < n_chunks)
                def _():
                    start_in(i + LOOKAHEAD, (b + LOOKAHEAD) % NBUF)

                @pl.when(i >= NBUF)
                def _():
                    wait_out(i - NBUF, b)

                for r in range(CHUNK):
                    @plsc.parallel_loop(0, E // LANES, unroll=8)
                    def col(j):
                        sl = pl.ds(j * LANES, LANES)
                        o_v[b, r, sl] = gat_v[b, r, sl] + tok_v[b, r, sl]

                start_out(i, b)

        for k in range(NBUF):
            wait_out(n_chunks - NBUF + k, (n_chunks - NBUF + k) % NBUF)

    return body


def kernel(token_embedding, pos, pos_table):
    B, L, E = token_embedding.shape
    V = pos_table.shape[0]
    N = B * L
    tok = token_embedding.reshape(N, E)
    idx = pos.reshape(NW, N // (NW * CHUNK), CHUNK).astype(jnp.int32)
    out = _make_sc_kernel(N, E, V)(tok, idx, pos_table)
    return out.reshape(B, L, E)
